# baseline (speedup 1.0000x reference)
import functools

import jax
import jax.numpy as jnp
from jax import lax
from jax.experimental import pallas as pl

_BF = jnp.bfloat16
_F32 = jnp.float32


def _prep_body(x_ref, p1_ref, p2t_ref, xt_ref, h1_ref, h2_ref):
    xt = x_ref[...].astype(_BF).T
    xt_ref[...] = xt
    h1_ref[...] = lax.dot_general(
        p1_ref[...].astype(_BF),
        xt,
        (((0,), (0,)), ((), ())),
        preferred_element_type=_F32,
    ).astype(_BF)
    h2_ref[...] = jnp.dot(
        p2t_ref[...].astype(_BF), xt, preferred_element_type=_F32
    ).astype(_BF)


def _prep_call(x, p1, p2t):
    n_tok, d = x.shape
    k1 = p1.shape[1]
    k2 = p2t.shape[0]
    return pl.pallas_call(
        _prep_body,
        out_shape=(
            jax.ShapeDtypeStruct((d, n_tok), _BF),
            jax.ShapeDtypeStruct((k1, n_tok), _BF),
            jax.ShapeDtypeStruct((k2, n_tok), _BF),
        ),
    )(x, p1, p2t)


def _cluster_body(rhs_ref, w_ref, b_ref, o_ref, *, w_transposed):
    if w_transposed:
        acc = jnp.dot(
            w_ref[...].astype(_BF), rhs_ref[...], preferred_element_type=_F32
        )
    else:
        acc = lax.dot_general(
            w_ref[...].astype(_BF),
            rhs_ref[...],
            (((0,), (0,)), ((), ())),
            preferred_element_type=_F32,
        )
    o_ref[...] = acc + b_ref[...][:, None]


def _cluster_call(rhs, w, b, bn, w_transposed):
    k, n_tok = rhs.shape
    n_out = w.shape[0] if w_transposed else w.shape[1]
    if w_transposed:
        w_spec = pl.BlockSpec((bn, k), lambda j: (j, 0))
    else:
        w_spec = pl.BlockSpec((k, bn), lambda j: (0, j))
    return pl.pallas_call(
        functools.partial(_cluster_body, w_transposed=w_transposed),
        grid=(pl.cdiv(n_out, bn),),
        in_specs=[
            pl.BlockSpec((k, n_tok), lambda j: (0, 0)),
            w_spec,
            pl.BlockSpec((bn,), lambda j: (j,)),
        ],
        out_specs=pl.BlockSpec((bn, n_tok), lambda j: (j, 0)),
        out_shape=jax.ShapeDtypeStruct((n_out, n_tok), _F32),
    )(rhs, w, b)


def kernel(x, W0, b0, P1, W1, b1, P2, W2, b2):
    xt, h1t, h2t = _prep_call(x, P1, P2.T)
    lh = _cluster_call(xt, W0.T, b0, bn=512, w_transposed=True)
    lc1 = _cluster_call(h1t, W1.T, b1, bn=1024, w_transposed=True)
    lc2 = _cluster_call(h2t, W2, b2, bn=1024, w_transposed=False)
    return (lh.T, lc1.T, lc2.T)


# speedup vs baseline: 1.0067x; 1.0067x over previous
import functools

import jax
import jax.numpy as jnp
from jax import lax
from jax.experimental import pallas as pl

_BF = jnp.bfloat16
_F32 = jnp.float32


def _prep_body(x_ref, p1_ref, p2t_ref, xt_ref, h1_ref, h2_ref):
    xt = x_ref[...].astype(_BF).T
    xt_ref[...] = xt
    h1_ref[...] = lax.dot_general(
        p1_ref[...].astype(_BF),
        xt,
        (((0,), (0,)), ((), ())),
        preferred_element_type=_F32,
    ).astype(_BF)
    h2_ref[...] = jnp.dot(
        p2t_ref[...].astype(_BF), xt, preferred_element_type=_F32
    ).astype(_BF)


def _prep_call(x, p1, p2t):
    n_tok, d = x.shape
    k1 = p1.shape[1]
    k2 = p2t.shape[0]
    return pl.pallas_call(
        _prep_body,
        out_shape=(
            jax.ShapeDtypeStruct((d, n_tok), _BF),
            jax.ShapeDtypeStruct((k1, n_tok), _BF),
            jax.ShapeDtypeStruct((k2, n_tok), _BF),
        ),
    )(x, p1, p2t)


def _cluster_body(rhs_ref, w_ref, b_ref, o_ref, *, w_transposed):
    if w_transposed:
        acc = jnp.dot(
            w_ref[...].astype(_BF), rhs_ref[...], preferred_element_type=_F32
        )
    else:
        acc = lax.dot_general(
            w_ref[...].astype(_BF),
            rhs_ref[...],
            (((0,), (0,)), ((), ())),
            preferred_element_type=_F32,
        )
    o_ref[...] = acc + b_ref[...][:, None]


def _cluster_call(rhs, w, b, bn, w_transposed):
    k, n_tok = rhs.shape
    n_out = w.shape[0] if w_transposed else w.shape[1]
    if w_transposed:
        w_spec = pl.BlockSpec((bn, k), lambda j: (j, 0))
    else:
        w_spec = pl.BlockSpec((k, bn), lambda j: (0, j))
    return pl.pallas_call(
        functools.partial(_cluster_body, w_transposed=w_transposed),
        grid=(pl.cdiv(n_out, bn),),
        in_specs=[
            pl.BlockSpec((k, n_tok), lambda j: (0, 0)),
            w_spec,
            pl.BlockSpec((bn,), lambda j: (j,)),
        ],
        out_specs=pl.BlockSpec((bn, n_tok), lambda j: (j, 0)),
        out_shape=jax.ShapeDtypeStruct((n_out, n_tok), _F32),
    )(rhs, w, b)


def kernel(x, W0, b0, P1, W1, b1, P2, W2, b2):
    xt, h1t, h2t = _prep_call(x, P1, P2.T)
    lh = _cluster_call(xt, W0.T, b0, bn=1024, w_transposed=True)
    lc1 = _cluster_call(h1t, W1.T, b1, bn=2048, w_transposed=True)
    lc2 = _cluster_call(h2t, W2, b2, bn=2048, w_transposed=False)
    return (lh.T, lc1.T, lc2.T)


# speedup vs baseline: 1.0099x; 1.0031x over previous
import functools

import jax
import jax.numpy as jnp
from jax import lax
from jax.experimental import pallas as pl

_BF = jnp.bfloat16
_F32 = jnp.float32


def _prep_body(x_ref, p1_ref, p2t_ref, xt_ref, h1_ref, h2_ref):
    xt = x_ref[...].astype(_BF).T
    xt_ref[...] = xt
    h1_ref[...] = lax.dot_general(
        p1_ref[...].astype(_BF),
        xt,
        (((0,), (0,)), ((), ())),
        preferred_element_type=_F32,
    ).astype(_BF)
    h2_ref[...] = jnp.dot(
        p2t_ref[...].astype(_BF), xt, preferred_element_type=_F32
    ).astype(_BF)


def _prep_call(x, p1, p2t):
    n_tok, d = x.shape
    k1 = p1.shape[1]
    k2 = p2t.shape[0]
    return pl.pallas_call(
        _prep_body,
        out_shape=(
            jax.ShapeDtypeStruct((d, n_tok), _BF),
            jax.ShapeDtypeStruct((k1, n_tok), _BF),
            jax.ShapeDtypeStruct((k2, n_tok), _BF),
        ),
    )(x, p1, p2t)


def _cluster_body(rhs_ref, w_ref, b_ref, o_ref, *, w_transposed):
    if w_transposed:
        acc = jnp.dot(
            w_ref[...].astype(_BF), rhs_ref[...], preferred_element_type=_F32
        )
    else:
        acc = lax.dot_general(
            w_ref[...].astype(_BF),
            rhs_ref[...],
            (((0,), (0,)), ((), ())),
            preferred_element_type=_F32,
        )
    o_ref[...] = acc + b_ref[...][:, None]


def _cluster_call(rhs, w, b, bn, w_transposed):
    k, n_tok = rhs.shape
    n_out = w.shape[0] if w_transposed else w.shape[1]
    if w_transposed:
        w_spec = pl.BlockSpec((bn, k), lambda j: (j, 0))
    else:
        w_spec = pl.BlockSpec((k, bn), lambda j: (0, j))
    return pl.pallas_call(
        functools.partial(_cluster_body, w_transposed=w_transposed),
        grid=(pl.cdiv(n_out, bn),),
        in_specs=[
            pl.BlockSpec((k, n_tok), lambda j: (0, 0)),
            w_spec,
            pl.BlockSpec((bn,), lambda j: (j,)),
        ],
        out_specs=pl.BlockSpec((bn, n_tok), lambda j: (j, 0)),
        out_shape=jax.ShapeDtypeStruct((n_out, n_tok), _F32),
    )(rhs, w, b)


def kernel(x, W0, b0, P1, W1, b1, P2, W2, b2):
    xt, h1t, h2t = _prep_call(x, P1, P2.T)
    lh = _cluster_call(xt, W0.T, b0, bn=512, w_transposed=True)
    lc1 = _cluster_call(h1t, W1.T, b1, bn=1024, w_transposed=True)
    lc2 = _cluster_call(h2t, W2, b2, bn=2048, w_transposed=False)
    return (lh.T, lc1.T, lc2.T)


# speedup vs baseline: 1.0318x; 1.0217x over previous
import functools

import jax
import jax.numpy as jnp
from jax import lax
from jax.experimental import pallas as pl

_BF = jnp.bfloat16
_F32 = jnp.float32


def _prep_body(x_ref, p1_ref, p2t_ref, xt_ref, h1_ref, h2_ref):
    xt = x_ref[...].astype(_BF).T
    xt_ref[...] = xt
    h1_ref[...] = lax.dot_general(
        p1_ref[...].astype(_BF),
        xt,
        (((0,), (0,)), ((), ())),
        preferred_element_type=_F32,
    ).astype(_BF)
    h2_ref[...] = jnp.dot(
        p2t_ref[...].astype(_BF), xt, preferred_element_type=_F32
    ).astype(_BF)


def _prep_call(x, p1, p2t):
    n_tok, d = x.shape
    k1 = p1.shape[1]
    k2 = p2t.shape[0]
    return pl.pallas_call(
        _prep_body,
        out_shape=(
            jax.ShapeDtypeStruct((d, n_tok), _BF),
            jax.ShapeDtypeStruct((k1, n_tok), _BF),
            jax.ShapeDtypeStruct((k2, n_tok), _BF),
        ),
    )(x, p1, p2t)


def _cluster_body(rhs_ref, w_ref, b_ref, o_ref, *, w_transposed):
    if w_transposed:
        acc = jnp.dot(
            w_ref[...].astype(_BF), rhs_ref[...], preferred_element_type=_F32
        )
    else:
        acc = lax.dot_general(
            w_ref[...].astype(_BF),
            rhs_ref[...],
            (((0,), (0,)), ((), ())),
            preferred_element_type=_F32,
        )
    o_ref[...] = acc + b_ref[...][:, None]


def _cluster_call(rhs, w, b, bn, w_transposed):
    k, n_tok = rhs.shape
    n_out = w.shape[0] if w_transposed else w.shape[1]
    if w_transposed:
        w_spec = pl.BlockSpec((bn, k), lambda j: (j, 0))
    else:
        w_spec = pl.BlockSpec((k, bn), lambda j: (0, j))
    return pl.pallas_call(
        functools.partial(_cluster_body, w_transposed=w_transposed),
        grid=(pl.cdiv(n_out, bn),),
        in_specs=[
            pl.BlockSpec((k, n_tok), lambda j: (0, 0)),
            w_spec,
            pl.BlockSpec((bn,), lambda j: (j,)),
        ],
        out_specs=pl.BlockSpec((bn, n_tok), lambda j: (j, 0)),
        out_shape=jax.ShapeDtypeStruct((n_out, n_tok), _F32),
    )(rhs, w, b)


def _c2h_body(xt_ref, h2_ref, w2_ref, b2_ref, w0t_ref, b0_ref, o2_ref, oh_ref, *, nh):
    j = pl.program_id(0)
    o2_ref[...] = (
        lax.dot_general(
            w2_ref[...].astype(_BF),
            h2_ref[...],
            (((0,), (0,)), ((), ())),
            preferred_element_type=_F32,
        )
        + b2_ref[...][:, None]
    )

    @pl.when(j < nh)
    def _():
        oh_ref[...] = (
            jnp.dot(
                w0t_ref[...].astype(_BF),
                xt_ref[...],
                preferred_element_type=_F32,
            )
            + b0_ref[...][:, None]
        )


def _c2h_call(xt, h2t, w2, b2, w0t, b0, bn2, bnh):
    d, n_tok = xt.shape
    k2, n2 = w2.shape
    n0 = w0t.shape[0]
    nh = pl.cdiv(n0, bnh)
    nsteps = pl.cdiv(n2, bn2)
    assert nsteps >= nh
    clamp = nh - 1

    def _h_idx(j):
        return (jnp.minimum(j, clamp), 0)

    def _h_idx1(j):
        return (jnp.minimum(j, clamp),)

    return pl.pallas_call(
        functools.partial(_c2h_body, nh=nh),
        grid=(nsteps,),
        in_specs=[
            pl.BlockSpec((d, n_tok), lambda j: (0, 0)),
            pl.BlockSpec((k2, n_tok), lambda j: (0, 0)),
            pl.BlockSpec((k2, bn2), lambda j: (0, j)),
            pl.BlockSpec((bn2,), lambda j: (j,)),
            pl.BlockSpec((bnh, d), _h_idx),
            pl.BlockSpec((bnh,), _h_idx1),
        ],
        out_specs=(
            pl.BlockSpec((bn2, n_tok), lambda j: (j, 0)),
            pl.BlockSpec((bnh, n_tok), _h_idx),
        ),
        out_shape=(
            jax.ShapeDtypeStruct((n2, n_tok), _F32),
            jax.ShapeDtypeStruct((n0, n_tok), _F32),
        ),
    )(xt, h2t, w2, b2, w0t, b0)


def kernel(x, W0, b0, P1, W1, b1, P2, W2, b2):
    xt, h1t, h2t = _prep_call(x, P1, P2.T)
    lc1 = _cluster_call(h1t, W1.T, b1, bn=1024, w_transposed=True)
    lc2, lh = _c2h_call(xt, h2t, W2, b2, W0.T, b0, bn2=2048, bnh=512)
    return (lh.T, lc1.T, lc2.T)
